# Initial kernel scaffold; baseline (speedup 1.0000x reference)
import jax
import jax.numpy as jnp
from jax.experimental import pallas as pl


def kernel(x, mask, table):
    raise NotImplementedError("write your pallas kernel here")


# speedup vs baseline: 1.8804x; 1.8804x over previous
import functools

import jax
import jax.numpy as jnp
from jax import lax
from jax.experimental import pallas as pl
from jax.experimental.pallas import tpu as pltpu
from jax.experimental.pallas import tpu_sc as plsc

D = 64
HIST = 50
LANES = 16
NC = 2
NS = 16
NW = NC * NS


def _tower_body(x_hbm, mask_hbm, table_hbm, out_hbm,
                idx_v, mask_v, rows_v, out_v, sem, bpw):
    wid = lax.axis_index("s") * NC + lax.axis_index("c")
    base = wid * bpw

    pltpu.sync_copy(x_hbm.at[pl.ds(base, bpw)], idx_v)
    pltpu.sync_copy(mask_hbm.at[pl.ds(base, bpw)], mask_v)

    group_starts = (0, 16, 32, HIST - LANES)

    def body(b, carry):
        pltpu.async_copy(table_hbm.at[idx_v.at[b]], rows_v, sem).wait()
        acc = [jnp.zeros((LANES,), jnp.float32) for _ in range(D // LANES)]
        cnt = jnp.zeros((LANES,), jnp.float32)
        mgroups = [mask_v[b, pl.ds(s, LANES)] for s in group_starts]
        for l in range(HIST):
            g, lane = (divmod(l, LANES) if l < 48
                       else (3, l - (HIST - LANES)))
            mv = jnp.full((LANES,), mgroups[g][lane], dtype=jnp.float32)
            cnt = cnt + mv
            for j in range(D // LANES):
                acc[j] = acc[j] + rows_v[l, j * LANES:(j + 1) * LANES] * mv
        inv = 1.0 / jnp.maximum(cnt, 1.0)
        for j in range(D // LANES):
            out_v[b, j * LANES:(j + 1) * LANES] = acc[j] * inv
        return carry

    lax.fori_loop(0, bpw, body, 0)
    pltpu.sync_copy(out_v, out_hbm.at[pl.ds(base, bpw)])


def kernel(x, mask, table):
    batch = x.shape[0]
    bpw = batch // NW
    mesh = plsc.VectorSubcoreMesh(core_axis_name="c", subcore_axis_name="s")

    tower = functools.partial(
        pl.kernel,
        out_type=jax.ShapeDtypeStruct((batch, D), jnp.float32),
        mesh=mesh,
        scratch_types=[
            pltpu.VMEM((bpw, HIST), jnp.int32),
            pltpu.VMEM((bpw, HIST), jnp.float32),
            pltpu.VMEM((HIST, D), jnp.float32),
            pltpu.VMEM((bpw, D), jnp.float32),
            pltpu.SemaphoreType.DMA,
        ],
        compiler_params=pltpu.CompilerParams(use_tc_tiling_on_sc=False),
    )(functools.partial(_tower_body, bpw=bpw))

    return tower(x, mask, table)


# speedup vs baseline: 2.6348x; 1.4012x over previous
import functools

import jax
import jax.numpy as jnp
from jax import lax
from jax.experimental import pallas as pl
from jax.experimental.pallas import tpu as pltpu
from jax.experimental.pallas import tpu_sc as plsc

D = 64
HIST = 50
LANES = 16
NC = 2
NS = 16
NW = NC * NS


NBUF = 4


def _tower_body(x_hbm, mask_hbm, table_hbm, out_hbm,
                idx_v, mask_v, rows0, rows1, rows2, rows3, out_v,
                sem0, sem1, sem2, sem3, bpw):
    wid = lax.axis_index("s") * NC + lax.axis_index("c")
    base = wid * bpw
    rows = (rows0, rows1, rows2, rows3)
    sems = (sem0, sem1, sem2, sem3)

    pltpu.sync_copy(x_hbm.at[pl.ds(base, bpw)], idx_v)
    pltpu.sync_copy(mask_hbm.at[pl.ds(base, bpw)], mask_v)

    group_starts = (0, 16, 32, HIST - LANES)

    for ph in range(NBUF):
        pltpu.async_copy(table_hbm.at[idx_v.at[ph]], rows[ph], sems[ph])

    def compute_row(b, rbuf):
        acc = [jnp.zeros((LANES,), jnp.float32) for _ in range(D // LANES)]
        cnt = jnp.zeros((LANES,), jnp.float32)
        mgroups = [mask_v[b, pl.ds(s, LANES)] for s in group_starts]
        for l in range(HIST):
            g, lane = (divmod(l, LANES) if l < 48
                       else (3, l - (HIST - LANES)))
            mv = jnp.full((LANES,), mgroups[g][lane], dtype=jnp.float32)
            cnt = cnt + mv
            for j in range(D // LANES):
                acc[j] = acc[j] + rbuf[l, j * LANES:(j + 1) * LANES] * mv
        inv = 1.0 / jnp.maximum(cnt, 1.0)
        for j in range(D // LANES):
            out_v[b, j * LANES:(j + 1) * LANES] = acc[j] * inv

    def body(g, carry):
        for ph in range(NBUF):
            b = g * NBUF + ph
            pltpu.make_async_copy(
                table_hbm.at[pl.ds(0, HIST)], rows[ph], sems[ph]).wait()
            compute_row(b, rows[ph])
            nxt = b + NBUF

            @pl.when(nxt < bpw)
            def _():
                pltpu.async_copy(table_hbm.at[idx_v.at[nxt]], rows[ph],
                                 sems[ph])
        return carry

    lax.fori_loop(0, bpw // NBUF, body, 0)
    pltpu.sync_copy(out_v, out_hbm.at[pl.ds(base, bpw)])


def kernel(x, mask, table):
    batch = x.shape[0]
    bpw = batch // NW
    mesh = plsc.VectorSubcoreMesh(core_axis_name="c", subcore_axis_name="s")

    tower = functools.partial(
        pl.kernel,
        out_type=jax.ShapeDtypeStruct((batch, D), jnp.float32),
        mesh=mesh,
        scratch_types=[
            pltpu.VMEM((bpw, HIST), jnp.int32),
            pltpu.VMEM((bpw, HIST), jnp.float32),
            pltpu.VMEM((HIST, D), jnp.float32),
            pltpu.VMEM((HIST, D), jnp.float32),
            pltpu.VMEM((HIST, D), jnp.float32),
            pltpu.VMEM((HIST, D), jnp.float32),
            pltpu.VMEM((bpw, D), jnp.float32),
            pltpu.SemaphoreType.DMA,
            pltpu.SemaphoreType.DMA,
            pltpu.SemaphoreType.DMA,
            pltpu.SemaphoreType.DMA,
        ],
        compiler_params=pltpu.CompilerParams(use_tc_tiling_on_sc=False),
    )(functools.partial(_tower_body, bpw=bpw))

    return tower(x, mask, table)


# speedup vs baseline: 2.6493x; 1.0055x over previous
import functools

import jax
import jax.numpy as jnp
from jax import lax
from jax.experimental import pallas as pl
from jax.experimental.pallas import tpu as pltpu
from jax.experimental.pallas import tpu_sc as plsc

D = 64
DP = 128
HIST = 50
HISTP = 56
LANES = 16
NC = 2
NS = 16
NW = NC * NS
NBUF = 4


def _tower_body(xp_hbm, ms_hbm, table_hbm, out_hbm,
                xp_v, ms_v, rows0, rows1, rows2, rows3, out_v,
                sem0, sem1, sem2, sem3, bpw):
    wid = lax.axis_index("s") * NC + lax.axis_index("c")
    rows = (rows0, rows1, rows2, rows3)
    sems = (sem0, sem1, sem2, sem3)
    in_base = wid * bpw * HISTP
    out_base = wid * bpw * D

    pltpu.sync_copy(xp_hbm.at[pl.ds(in_base, bpw * HISTP)], xp_v)
    pltpu.sync_copy(ms_hbm.at[pl.ds(in_base, bpw * HISTP)], ms_v)

    group_starts = (0, 16, 32, 40)

    def issue(b, ph):
        pltpu.async_copy(table_hbm.at[xp_v.at[pl.ds(b * HISTP, HIST)]],
                         rows[ph], sems[ph])

    for ph in range(NBUF):
        issue(ph, ph)

    def compute_row(b, rbuf):
        acc = [jnp.zeros((LANES,), jnp.float32) for _ in range(D // LANES)]
        cnt = jnp.zeros((LANES,), jnp.float32)
        mgroups = [ms_v[pl.ds(b * HISTP + s, LANES)] for s in group_starts]
        for l in range(HIST):
            g, lane = (divmod(l, LANES) if l < 48 else (3, l - 40))
            w = jnp.full((LANES,), mgroups[g][lane], dtype=jnp.float32)
            cnt = cnt + w
            for j in range(D // LANES):
                acc[j] = acc[j] + rbuf[l, j * LANES:(j + 1) * LANES] * w
        inv = 1.0 / jnp.maximum(cnt, 1.0)
        for j in range(D // LANES):
            out_v[pl.ds(b * D + j * LANES, LANES)] = acc[j] * inv

    def body(g, carry):
        for ph in range(NBUF):
            b = g * NBUF + ph
            pltpu.make_async_copy(
                table_hbm.at[xp_v.at[pl.ds(b * HISTP, HIST)]],
                rows[ph], sems[ph]).wait()
            compute_row(b, rows[ph])
            nxt = b + NBUF

            @pl.when(nxt < bpw)
            def _():
                issue(nxt, ph)
        return carry

    lax.fori_loop(0, bpw // NBUF, body, 0)
    pltpu.sync_copy(out_v, out_hbm.at[pl.ds(out_base, bpw * D)])


def kernel(x, mask, table):
    batch = x.shape[0]
    vocab = table.shape[0]
    bpw = batch // NW
    t2 = jnp.pad(table, ((0, 0), (0, DP - D)))
    xp = jnp.pad(x, ((0, 0), (0, HISTP - HIST))).reshape(-1)
    ms = jnp.pad(mask, ((0, 0), (0, HISTP - HIST))).reshape(-1)

    mesh = plsc.VectorSubcoreMesh(core_axis_name="c", subcore_axis_name="s")
    tower = functools.partial(
        pl.kernel,
        out_type=jax.ShapeDtypeStruct((batch * D,), jnp.float32),
        mesh=mesh,
        scratch_types=[
            pltpu.VMEM((bpw * HISTP,), jnp.int32),
            pltpu.VMEM((bpw * HISTP,), jnp.float32),
            pltpu.VMEM((HIST, DP), jnp.float32),
            pltpu.VMEM((HIST, DP), jnp.float32),
            pltpu.VMEM((HIST, DP), jnp.float32),
            pltpu.VMEM((HIST, DP), jnp.float32),
            pltpu.VMEM((bpw * D,), jnp.float32),
            pltpu.SemaphoreType.DMA,
            pltpu.SemaphoreType.DMA,
            pltpu.SemaphoreType.DMA,
            pltpu.SemaphoreType.DMA,
        ],
        compiler_params=pltpu.CompilerParams(use_tc_tiling_on_sc=True),
    )(functools.partial(_tower_body, bpw=bpw))

    return tower(xp, ms, t2).reshape(batch, D)
